# baseline (speedup 1.0000x reference)
import functools

import jax
import jax.numpy as jnp
from jax import lax
from jax.experimental import pallas as pl
from jax.experimental.pallas import tpu as pltpu
from jax.experimental.pallas import tpu_sc as plsc

D = 64
L = 16
NC = 2
NS = 16
NW = NC * NS
BT = 128
DT = D // 8
NBUF = 2


def _build(batch, seq):
    assert batch == NW * BT
    assert seq % 8 == 0 and (seq // 8) % 1 == 0
    st = seq // 8
    assert seq % NBUF == 0 and seq >= 2 * NBUF

    mesh = plsc.VectorSubcoreMesh(core_axis_name="c", subcore_axis_name="s")

    @functools.partial(
        pl.kernel,
        out_type=jax.ShapeDtypeStruct((seq, DT, NW, 8, BT), jnp.float32),
        mesh=mesh,
        scratch_types=[
            pltpu.VMEM((st, 1, 8, BT), jnp.int32),
            pltpu.VMEM((st, 1, 8, BT), jnp.int32),
            pltpu.VMEM((NBUF, BT, D), jnp.float32),
            pltpu.VMEM((NBUF, BT, D), jnp.float32),
            pltpu.VMEM((NBUF, DT, 1, 8, BT), jnp.float32),
            pltpu.SemaphoreType.DMA((NBUF,)),
            pltpu.SemaphoreType.DMA((NBUF,)),
        ],
        compiler_params=pltpu.CompilerParams(
            use_tc_tiling_on_sc=False, needs_layout_passes=False),
    )
    def kern(tok_idx_hbm, diac_idx_hbm, tok_tab_hbm, diac_tab_hbm, out_hbm,
             it_v, id_v, tr_v, dr_v, ob_v, gsems, wsems):
        wid = lax.axis_index("s") * NC + lax.axis_index("c")

        pltpu.sync_copy(tok_idx_hbm.at[:, pl.ds(wid, 1)], it_v)
        pltpu.sync_copy(diac_idx_hbm.at[:, pl.ds(wid, 1)], id_v)

        def gather_descs(s, b):
            tr8 = s // 8
            r = lax.rem(s, 8)
            return [
                pltpu.make_async_copy(
                    tok_tab_hbm.at[it_v.at[tr8, 0, r]], tr_v.at[b], gsems.at[b]),
                pltpu.make_async_copy(
                    diac_tab_hbm.at[id_v.at[tr8, 0, r]], dr_v.at[b], gsems.at[b]),
            ]

        def issue_gathers(s, b):
            for d in gather_descs(s, b):
                d.start()

        def wait_gathers(s, b):
            for d in gather_descs(s, b):
                d.wait()

        def write_desc(s, b):
            return pltpu.make_async_copy(
                ob_v.at[b], out_hbm.at[s, pl.ds(0, DT), pl.ds(wid, 1)],
                wsems.at[b])

        bvecs = [lax.iota(jnp.int32, L) + L * k for k in range(BT // L)]

        def add_group(b):
            @pl.loop(0, DT)
            def _(dt):
                for dr in range(8):
                    d = dt * 8 + dr
                    dvec = jnp.full((L,), d, jnp.int32)
                    for k in range(BT // L):
                        acc = (plsc.load_gather(tr_v.at[b], [bvecs[k], dvec])
                               + plsc.load_gather(dr_v.at[b], [bvecs[k], dvec]))
                        ob_v[b, dt, 0, dr, pl.ds(k * L, L)] = acc

        for b in range(NBUF):
            issue_gathers(b, b)

        @pl.loop(0, seq - NBUF, step=NBUF)
        def _(s0):
            for b in range(NBUF):
                s = s0 + b
                wait_gathers(s, b)

                @pl.when(s0 >= NBUF)
                def _():
                    write_desc(s - NBUF, b).wait()

                add_group(b)
                write_desc(s, b).start()
                issue_gathers(s + NBUF, b)

        for b in range(NBUF):
            s = seq - NBUF + b
            wait_gathers(s, b)
            write_desc(s - NBUF, b).wait()
            add_group(b)
            write_desc(s, b).start()
        for b in range(NBUF):
            s = seq - NBUF + b
            write_desc(s, b).wait()

    return kern


_kern = _build(4096, 200)


def kernel(token_inputs, diac_inputs, token_table, diac_table):
    B, S = token_inputs.shape
    def fmt(idx):
        return (idx.T.reshape(S // 8, 8, NW, BT).transpose(0, 2, 1, 3))

    out = _kern(fmt(token_inputs), fmt(diac_inputs), token_table, diac_table)
    return out.transpose(2, 4, 0, 1, 3).reshape(B, S, D)


# speedup vs baseline: 5.6629x; 5.6629x over previous
import functools

import jax
import jax.numpy as jnp
from jax import lax
from jax.experimental import pallas as pl
from jax.experimental.pallas import tpu as pltpu
from jax.experimental.pallas import tpu_sc as plsc

D = 64
L = 16
NC = 2
NS = 16
NW = NC * NS
BT = 128
DT = D // 8
NBUF = 2


def _build(batch, seq):
    assert batch == NW * BT
    assert seq % 8 == 0 and (seq // 8) % 1 == 0
    st = seq // 8
    assert seq % NBUF == 0 and seq >= 2 * NBUF

    mesh = plsc.VectorSubcoreMesh(core_axis_name="c", subcore_axis_name="s")

    @functools.partial(
        pl.kernel,
        out_type=jax.ShapeDtypeStruct((seq, DT, NW, 8, BT), jnp.float32),
        mesh=mesh,
        scratch_types=[
            pltpu.VMEM((st, 1, 8, BT), jnp.int32),
            pltpu.VMEM((st, 1, 8, BT), jnp.int32),
            pltpu.VMEM((NBUF, BT, D), jnp.float32),
            pltpu.VMEM((NBUF, BT, D), jnp.float32),
            pltpu.VMEM((NBUF, DT, 1, 8, BT), jnp.float32),
            pltpu.SemaphoreType.DMA((NBUF,)),
            pltpu.SemaphoreType.DMA((NBUF,)),
        ],
        compiler_params=pltpu.CompilerParams(
            use_tc_tiling_on_sc=False, needs_layout_passes=False),
    )
    def kern(tok_idx_hbm, diac_idx_hbm, tok_tab_hbm, diac_tab_hbm, out_hbm,
             it_v, id_v, tr_v, dr_v, ob_v, gsems, wsems):
        wid = lax.axis_index("s") * NC + lax.axis_index("c")

        pltpu.sync_copy(tok_idx_hbm.at[:, pl.ds(wid, 1)], it_v)
        pltpu.sync_copy(diac_idx_hbm.at[:, pl.ds(wid, 1)], id_v)

        def gather_descs(s, b):
            tr8 = s // 8
            r = lax.rem(s, 8)
            return [
                pltpu.make_async_copy(
                    tok_tab_hbm.at[it_v.at[tr8, 0, r]], tr_v.at[b], gsems.at[b]),
                pltpu.make_async_copy(
                    diac_tab_hbm.at[id_v.at[tr8, 0, r]], dr_v.at[b], gsems.at[b]),
            ]

        def issue_gathers(s, b):
            for d in gather_descs(s, b):
                d.start()

        def wait_gathers(s, b):
            for d in gather_descs(s, b):
                d.wait()

        def write_desc(s, b):
            return pltpu.make_async_copy(
                ob_v.at[b], out_hbm.at[s, pl.ds(0, DT), pl.ds(wid, 1)],
                wsems.at[b])

        bvecs = [lax.iota(jnp.int32, L) + L * k for k in range(BT // L)]

        def add_group(b):
            @functools.partial(plsc.parallel_loop, 0, D, unroll=2)
            def _(d):
                dt = d // 8
                dr = lax.rem(d, 8)
                dvec = jnp.full((L,), d, jnp.int32)
                for k in range(BT // L):
                    acc = (plsc.load_gather(tr_v.at[b], [bvecs[k], dvec])
                           + plsc.load_gather(dr_v.at[b], [bvecs[k], dvec]))
                    ob_v[b, dt, 0, dr, pl.ds(k * L, L)] = acc

        for b in range(NBUF):
            issue_gathers(b, b)

        @pl.loop(0, seq - NBUF, step=NBUF)
        def _(s0):
            for b in range(NBUF):
                s = s0 + b
                wait_gathers(s, b)

                @pl.when(s0 >= NBUF)
                def _():
                    write_desc(s - NBUF, b).wait()

                add_group(b)
                write_desc(s, b).start()
                issue_gathers(s + NBUF, b)

        for b in range(NBUF):
            s = seq - NBUF + b
            wait_gathers(s, b)
            write_desc(s - NBUF, b).wait()
            add_group(b)
            write_desc(s, b).start()
        for b in range(NBUF):
            s = seq - NBUF + b
            write_desc(s, b).wait()

    return kern


_kern = _build(4096, 200)


def kernel(token_inputs, diac_inputs, token_table, diac_table):
    B, S = token_inputs.shape
    def fmt(idx):
        return (idx.T.reshape(S // 8, 8, NW, BT).transpose(0, 2, 1, 3))

    out = _kern(fmt(token_inputs), fmt(diac_inputs), token_table, diac_table)
    return out.transpose(2, 4, 0, 1, 3).reshape(B, S, D)
